# Initial kernel scaffold; baseline (speedup 1.0000x reference)
import jax
import jax.numpy as jnp
from jax.experimental import pallas as pl


def kernel(user_feat, item_feat, edge_feature, pos_src, pos_dst, neg_src, neg_dst, user_nb, item_nb, W_pu, b_pu, W_pi, b_pi, Wsu0, Wnu0, Wsu1, Wnu1, Wsi0, Wni0, Wsi1, Wni1, W_lin, b_lin):
    raise NotImplementedError("write your pallas kernel here")


# speedup vs baseline: 2.6681x; 2.6681x over previous
import jax
import jax.numpy as jnp
from jax import lax
from jax.experimental import pallas as pl
from jax.experimental.pallas import tpu as pltpu
from jax.experimental.pallas import tpu_sc as plsc

K0 = 10
K1 = 5
D = 128
NC = 2
NS = 16
NW = NC * NS
NE = 4096
EPW = NE // NW
C = 32
NCH = EPW // C
NB1 = C * K0
NU2 = NB1 * K1
G = 80
NG = NU2 // G
MPG = G // K1
EB = 256
_SEG1 = ((0, 128), (128, 128), (256, 64))
_SEG2 = tuple((o * 128, 128) for o in range(12)) + ((1536, 64),)


def _proj_body(x_ref, w_ref, b_ref, o_ref):
  o_ref[...] = jax.nn.sigmoid(
      jnp.dot(x_ref[...], w_ref[...], preferred_element_type=jnp.float32)
      + b_ref[...])


def _project(x, W, b):
  N, Din = x.shape
  PD = W.shape[1]
  blk = 1000
  return pl.pallas_call(
      _proj_body,
      grid=(N // blk,),
      in_specs=[
          pl.BlockSpec((blk, Din), lambda i: (i, 0)),
          pl.BlockSpec((Din, PD), lambda i: (0, 0)),
          pl.BlockSpec((1, PD), lambda i: (0, 0)),
      ],
      out_specs=pl.BlockSpec((blk, PD), lambda i: (i, 0)),
      out_shape=jax.ShapeDtypeStruct((N, PD), jnp.float32),
  )(x, W, b.reshape(1, PD))


def _sc_body(up, ip, unbf, inbf, src, dst,
             h0u, h1u, s2u, h0i, h1i, s2i, u1st,
             srcv, idx1v, u1v, h0v, h1v, iav, ibv, h2v, m2v,
             sem0, sem1, sem2):
  cid = lax.axis_index("c")
  sid = lax.axis_index("s")
  wid = cid * NS + sid
  iota = lax.iota(jnp.int32, 16)
  vK0 = jnp.full((16,), K0, jnp.int32)
  vK1 = jnp.full((16,), K1, jnp.int32)

  for (ev, nbAf, nbBf, featA, featB, h0o, h1o, s2o) in (
      (src, unbf, inbf, up, ip, h0u, h1u, s2u),
      (dst, inbf, unbf, ip, up, h0i, h1i, s2i)):
    pltpu.sync_copy(ev.at[pl.ds(wid * EPW, EPW)], srcv)
    for c in range(NCH):
      e0 = wid * EPW + c * C
      p0 = (wid * NCH + c) * NB1
      idx_c = srcv.at[pl.ds(c * C, C)]
      cp_h0 = pltpu.async_copy(featA.at[idx_c], h0v, sem0)

      for i in range(NB1 // 16):
        k0v = i // 2
        sl = srcv[pl.ds(c * C + (i % 2) * 16, 16)]
        idx1v[pl.ds(i * 16, 16)] = sl * vK0 + jnp.full((16,), k0v, jnp.int32)
      cps = [pltpu.async_copy(nbAf.at[idx1v.at[pl.ds(o, n)]],
                              u1v.at[pl.ds(o, n)], sem1) for (o, n) in _SEG1]
      cp_h0.wait()
      pltpu.sync_copy(h0v, h0o.at[pl.ds(e0, C)])
      for cp in cps:
        cp.wait()

      cps = [pltpu.async_copy(featB.at[u1v.at[pl.ds(o, n)]],
                              h1v.at[pl.ds(o, n)], sem0) for (o, n) in _SEG1]
      pltpu.sync_copy(u1v, u1st.at[pl.ds(p0, NB1)])

      def build_rep(i, _):
        base = pl.multiple_of(i * 16, 16)
        q = base + iota
        iav[pl.ds(base, 16)] = lax.div(q, vK1) + jnp.full((16,), p0, jnp.int32)
        return _
      lax.fori_loop(0, NU2 // 16, build_rep, None)
      cps2 = [pltpu.async_copy(u1st.at[iav.at[pl.ds(o, n)]],
                               ibv.at[pl.ds(o, n)], sem1) for (o, n) in _SEG2]
      for cp in cps:
        cp.wait()
      for k0 in range(K0):
        pltpu.sync_copy(h1v.at[pl.ds(k0 * C, C)], h1o.at[k0, pl.ds(e0, C)])
      for cp in cps2:
        cp.wait()

      def build_idx2(i, _):
        base = pl.multiple_of(i * 16, 16)
        q = base + iota
        iav[pl.ds(base, 16)] = (ibv[pl.ds(base, 16)] * vK0
                                + lax.rem(q, vK1))
        return _
      lax.fori_loop(0, NU2 // 16, build_idx2, None)
      cps = [pltpu.async_copy(nbBf.at[iav.at[pl.ds(o, n)]],
                              ibv.at[pl.ds(o, n)], sem1) for (o, n) in _SEG2]
      for cp in cps:
        cp.wait()

      def do_group(g, _):
        gbase = pl.multiple_of(g * G, G)
        pltpu.async_copy(featA.at[ibv.at[pl.ds(gbase, G)]], h2v, sem2).wait()

        def accum(m, _):
          r = m * K1
          mrow = g * MPG + m
          for v in range(D // 16):
            sl = pl.ds(v * 16, 16)
            s = (h2v[r, sl] + h2v[r + 1, sl] + h2v[r + 2, sl]
                 + h2v[r + 3, sl] + h2v[r + 4, sl])
            m2v[mrow, sl] = s
          return _
        lax.fori_loop(0, MPG, accum, None)
        return _
      lax.fori_loop(0, NG, do_group, None)

      for k0 in range(K0):
        pltpu.sync_copy(m2v.at[pl.ds(k0 * C, C)], s2o.at[k0, pl.ds(e0, C)])


def _sc_gather(up, ip, unbf, inbf, srcp, dstp):
  mesh = plsc.VectorSubcoreMesh(core_axis_name="c", subcore_axis_name="s",
                                num_cores=NC, num_subcores=NS)
  edge_t = jax.ShapeDtypeStruct((NE, D), jnp.float32)
  hop_t = jax.ShapeDtypeStruct((K0, NE, D), jnp.float32)
  f = pl.kernel(
      _sc_body,
      out_type=[edge_t, hop_t, hop_t, edge_t, hop_t, hop_t,
                jax.ShapeDtypeStruct((NE * K0,), jnp.int32)],
      mesh=mesh,
      scratch_types=[
          pltpu.VMEM((EPW,), jnp.int32),
          pltpu.VMEM((NB1,), jnp.int32),
          pltpu.VMEM((NB1,), jnp.int32),
          pltpu.VMEM((C, D), jnp.float32),
          pltpu.VMEM((NB1, D), jnp.float32),
          pltpu.VMEM((NU2,), jnp.int32),
          pltpu.VMEM((NU2,), jnp.int32),
          pltpu.VMEM((G, D), jnp.float32),
          pltpu.VMEM((NB1, D), jnp.float32),
          pltpu.SemaphoreType.DMA,
          pltpu.SemaphoreType.DMA,
          pltpu.SemaphoreType.DMA,
      ],
  )
  return f(up, ip, unbf, inbf, srcp, dstp)


def _gnn_body(h0u, h1u, s2u, h0i, h1i, s2i,
              wsu0, wnu0, wsu1, wnu1, wsi0, wni0, wsi1, wni1,
              wlin, blin, o_ref):
  def side(h0r, h1r, s2r, ws0, wn0, ws1, wn1):
    h0 = h0r[...]
    m1 = h1r[0]
    for k in range(1, K0):
      m1 = m1 + h1r[k]
    g0 = jax.nn.relu(
        jnp.dot(h0, ws0[...], preferred_element_type=jnp.float32)
        + jnp.dot(m1 * (1.0 / K0), wn0[...], preferred_element_type=jnp.float32))
    mg1 = None
    for k in range(K0):
      g1k = jax.nn.relu(
          jnp.dot(h1r[k], ws0[...], preferred_element_type=jnp.float32)
          + jnp.dot(s2r[k] * (1.0 / K1), wn0[...],
                    preferred_element_type=jnp.float32))
      mg1 = g1k if mg1 is None else mg1 + g1k
    return (jnp.dot(g0, ws1[...], preferred_element_type=jnp.float32)
            + jnp.dot(mg1 * (1.0 / K0), wn1[...],
                      preferred_element_type=jnp.float32))

  uh = side(h0u, h1u, s2u, wsu0, wnu0, wsu1, wnu1)
  ih = side(h0i, h1i, s2i, wsi0, wni0, wsi1, wni1)
  pred = uh * ih
  o_ref[...] = (jnp.dot(pred, wlin[...], preferred_element_type=jnp.float32)
                + blin[...])


def _gnn(h0u, h1u, s2u, h0i, h1i, s2i, weights, wlin_pad, blin):
  wspec = pl.BlockSpec((D, D), lambda i: (0, 0))
  return pl.pallas_call(
      _gnn_body,
      grid=(NE // EB,),
      in_specs=[
          pl.BlockSpec((EB, D), lambda i: (i, 0)),
          pl.BlockSpec((K0, EB, D), lambda i: (0, i, 0)),
          pl.BlockSpec((K0, EB, D), lambda i: (0, i, 0)),
          pl.BlockSpec((EB, D), lambda i: (i, 0)),
          pl.BlockSpec((K0, EB, D), lambda i: (0, i, 0)),
          pl.BlockSpec((K0, EB, D), lambda i: (0, i, 0)),
      ] + [wspec] * 9 + [pl.BlockSpec((1, D), lambda i: (0, 0))],
      out_specs=pl.BlockSpec((EB, D), lambda i: (i, 0)),
      out_shape=jax.ShapeDtypeStruct((NE, D), jnp.float32),
  )(h0u, h1u, s2u, h0i, h1i, s2i, *weights, wlin_pad, blin)


def kernel(user_feat, item_feat, edge_feature, pos_src, pos_dst, neg_src,
           neg_dst, user_nb, item_nb, W_pu, b_pu, W_pi, b_pi, Wsu0, Wnu0,
           Wsu1, Wnu1, Wsi0, Wni0, Wsi1, Wni1, W_lin, b_lin):
  up = _project(user_feat, W_pu, b_pu)
  ip = _project(item_feat, W_pi, b_pi)
  B = pos_src.shape[0] + neg_src.shape[0]
  src = jnp.concatenate([pos_src, neg_src]).astype(jnp.int32)
  dst = jnp.concatenate([pos_dst, neg_dst]).astype(jnp.int32)
  srcp = jnp.pad(src, (0, NE - B))
  dstp = jnp.pad(dst, (0, NE - B))
  unbf = user_nb.astype(jnp.int32).reshape(-1)
  inbf = item_nb.astype(jnp.int32).reshape(-1)
  h0u, h1u, s2u, h0i, h1i, s2i, _ = _sc_gather(up, ip, unbf, inbf, srcp, dstp)
  wlin_pad = jnp.pad(W_lin, ((0, 0), (0, D - 1)))
  blin = jnp.broadcast_to(b_lin.reshape(1, 1), (1, D))
  weights = (Wsu0, Wnu0, Wsu1, Wnu1, Wsi0, Wni0, Wsi1, Wni1)
  out = _gnn(h0u, h1u, s2u, h0i, h1i, s2i, weights, wlin_pad, blin)
  return out[:B, :1]


# speedup vs baseline: 3.6515x; 1.3686x over previous
import jax
import jax.numpy as jnp
from jax import lax
from jax.experimental import pallas as pl
from jax.experimental.pallas import tpu as pltpu
from jax.experimental.pallas import tpu_sc as plsc

K0 = 10
K1 = 5
D = 128
NC = 2
NS = 16
NW = NC * NS
NE = 4096
EPW = NE // NW
C = 32
NCH = EPW // C
NB1 = C * K0
NU2 = NB1 * K1
G = 80
NG = NU2 // G
MPG = G // K1
EB = 256
_SEG1 = ((0, 128), (128, 128), (256, 64))
_SEG2 = tuple((o * 128, 128) for o in range(12)) + ((1536, 64),)


def _proj_body(x_ref, w_ref, b_ref, o_ref):
  o_ref[...] = jax.nn.sigmoid(
      jnp.dot(x_ref[...], w_ref[...], preferred_element_type=jnp.float32)
      + b_ref[...])


def _project(x, W, b):
  N, Din = x.shape
  PD = W.shape[1]
  blk = 1000
  return pl.pallas_call(
      _proj_body,
      grid=(N // blk,),
      in_specs=[
          pl.BlockSpec((blk, Din), lambda i: (i, 0)),
          pl.BlockSpec((Din, PD), lambda i: (0, 0)),
          pl.BlockSpec((1, PD), lambda i: (0, 0)),
      ],
      out_specs=pl.BlockSpec((blk, PD), lambda i: (i, 0)),
      out_shape=jax.ShapeDtypeStruct((N, PD), jnp.float32),
  )(x, W, b.reshape(1, PD))


def _sc_body(up, ip, unbf, inbf, src, dst,
             h0u, h1u, s2u, h0i, h1i, s2i,
             srcv, idx1v, u1v, h0v, h1v, idx2v, u2v, h2a, h2b, m2v,
             semI, semF, semH, semA, semB):
  cid = lax.axis_index("c")
  sid = lax.axis_index("s")
  wid = cid * NS + sid
  iota = lax.iota(jnp.int32, 16)
  vK0 = jnp.full((16,), K0, jnp.int32)
  vK1 = jnp.full((16,), K1, jnp.int32)

  def accum(g, buf):
    def body(m, _):
      r = m * K1
      mrow = g * MPG + m
      for v in range(D // 16):
        sl = pl.ds(v * 16, 16)
        s = (buf[r, sl] + buf[r + 1, sl] + buf[r + 2, sl]
             + buf[r + 3, sl] + buf[r + 4, sl])
        m2v[mrow, sl] = s
      return _
    lax.fori_loop(0, MPG, body, None)

  for (ev, nbAf, nbBf, featA, featB, h0o, h1o, s2o) in (
      (src, unbf, inbf, up, ip, h0u, h1u, s2u),
      (dst, inbf, unbf, ip, up, h0i, h1i, s2i)):
    pltpu.sync_copy(ev.at[pl.ds(wid * EPW, EPW)], srcv)
    cp_h0 = pltpu.async_copy(featA.at[srcv], h0v, semH)
    for c in range(NCH):
      e0 = wid * EPW + c * C

      for i in range(NB1 // 16):
        k0v = i // 2
        sl = srcv[pl.ds(c * C + (i % 2) * 16, 16)]
        idx1v[pl.ds(i * 16, 16)] = sl * vK0 + jnp.full((16,), k0v, jnp.int32)
      cps = [pltpu.async_copy(nbAf.at[idx1v.at[pl.ds(o, n)]],
                              u1v.at[pl.ds(o, n)], semI) for (o, n) in _SEG1]
      if c == 0:
        cp_h0.wait()
        pltpu.sync_copy(h0v, h0o.at[pl.ds(wid * EPW, EPW)])
      for cp in cps:
        cp.wait()

      cpsF = [pltpu.async_copy(featB.at[u1v.at[pl.ds(o, n)]],
                               h1v.at[pl.ds(o, n)], semF) for (o, n) in _SEG1]

      def build_idx2(i, _):
        base = pl.multiple_of(i * 16, 16)
        q = base + iota
        p0 = base // 5
        v16 = u1v[pl.ds(p0, 16)]
        p = lax.div(q, vK1)
        d = p - jnp.full((16,), p0, jnp.int32)
        exp = jnp.where(
            d == 0, jnp.full((16,), v16[0], jnp.int32),
            jnp.where(d == 1, jnp.full((16,), v16[1], jnp.int32),
                      jnp.where(d == 2, jnp.full((16,), v16[2], jnp.int32),
                                jnp.full((16,), v16[3], jnp.int32))))
        idx2v[pl.ds(base, 16)] = exp * vK0 + (q - p * vK1)
        return _
      lax.fori_loop(0, NU2 // 16, build_idx2, None)
      cpsU = [pltpu.async_copy(nbBf.at[idx2v.at[pl.ds(o, n)]],
                               u2v.at[pl.ds(o, n)], semI) for (o, n) in _SEG2]
      for cp in cpsF:
        cp.wait()
      for k0 in range(K0):
        pltpu.sync_copy(h1v.at[pl.ds(k0 * C, C)], h1o.at[k0, pl.ds(e0, C)])
      for cp in cpsU:
        cp.wait()

      def issue_h2(g, buf, sem):
        gb = pl.multiple_of(g * G, G)
        return pltpu.async_copy(featA.at[u2v.at[pl.ds(gb, G)]], buf, sem)

      def wait_h2(g, buf, sem):
        gb = pl.multiple_of(g * G, G)
        pltpu.make_async_copy(featA.at[u2v.at[pl.ds(gb, G)]], buf, sem).wait()

      issue_h2(0, h2a, semA)

      def pipe(t, _):
        g0 = 2 * t
        issue_h2(g0 + 1, h2b, semB)
        wait_h2(g0, h2a, semA)
        accum(g0, h2a)

        @pl.when(t < NG // 2 - 1)
        def _():
          issue_h2(g0 + 2, h2a, semA)
        wait_h2(g0 + 1, h2b, semB)
        accum(g0 + 1, h2b)
        return _
      lax.fori_loop(0, NG // 2, pipe, None)

      for k0 in range(K0):
        pltpu.sync_copy(m2v.at[pl.ds(k0 * C, C)], s2o.at[k0, pl.ds(e0, C)])


def _sc_gather(up, ip, unbf, inbf, srcp, dstp):
  mesh = plsc.VectorSubcoreMesh(core_axis_name="c", subcore_axis_name="s",
                                num_cores=NC, num_subcores=NS)
  edge_t = jax.ShapeDtypeStruct((NE, D), jnp.float32)
  hop_t = jax.ShapeDtypeStruct((K0, NE, D), jnp.float32)
  f = pl.kernel(
      _sc_body,
      out_type=[edge_t, hop_t, hop_t, edge_t, hop_t, hop_t],
      mesh=mesh,
      scratch_types=[
          pltpu.VMEM((EPW,), jnp.int32),
          pltpu.VMEM((NB1,), jnp.int32),
          pltpu.VMEM((NB1 + 16,), jnp.int32),
          pltpu.VMEM((EPW, D), jnp.float32),
          pltpu.VMEM((NB1, D), jnp.float32),
          pltpu.VMEM((NU2,), jnp.int32),
          pltpu.VMEM((NU2,), jnp.int32),
          pltpu.VMEM((G, D), jnp.float32),
          pltpu.VMEM((G, D), jnp.float32),
          pltpu.VMEM((NB1, D), jnp.float32),
          pltpu.SemaphoreType.DMA,
          pltpu.SemaphoreType.DMA,
          pltpu.SemaphoreType.DMA,
          pltpu.SemaphoreType.DMA,
          pltpu.SemaphoreType.DMA,
      ],
  )
  return f(up, ip, unbf, inbf, srcp, dstp)


def _gnn_body(h0u, h1u, s2u, h0i, h1i, s2i,
              wsu0, wnu0, wsu1, wnu1, wsi0, wni0, wsi1, wni1,
              wlin, blin, o_ref):
  def side(h0r, h1r, s2r, ws0, wn0, ws1, wn1):
    h0 = h0r[...]
    m1 = h1r[0]
    for k in range(1, K0):
      m1 = m1 + h1r[k]
    g0 = jax.nn.relu(
        jnp.dot(h0, ws0[...], preferred_element_type=jnp.float32)
        + jnp.dot(m1 * (1.0 / K0), wn0[...], preferred_element_type=jnp.float32))
    mg1 = None
    for k in range(K0):
      g1k = jax.nn.relu(
          jnp.dot(h1r[k], ws0[...], preferred_element_type=jnp.float32)
          + jnp.dot(s2r[k] * (1.0 / K1), wn0[...],
                    preferred_element_type=jnp.float32))
      mg1 = g1k if mg1 is None else mg1 + g1k
    return (jnp.dot(g0, ws1[...], preferred_element_type=jnp.float32)
            + jnp.dot(mg1 * (1.0 / K0), wn1[...],
                      preferred_element_type=jnp.float32))

  uh = side(h0u, h1u, s2u, wsu0, wnu0, wsu1, wnu1)
  ih = side(h0i, h1i, s2i, wsi0, wni0, wsi1, wni1)
  pred = uh * ih
  o_ref[...] = (jnp.dot(pred, wlin[...], preferred_element_type=jnp.float32)
                + blin[...])


def _gnn(h0u, h1u, s2u, h0i, h1i, s2i, weights, wlin_pad, blin):
  wspec = pl.BlockSpec((D, D), lambda i: (0, 0))
  return pl.pallas_call(
      _gnn_body,
      grid=(NE // EB,),
      in_specs=[
          pl.BlockSpec((EB, D), lambda i: (i, 0)),
          pl.BlockSpec((K0, EB, D), lambda i: (0, i, 0)),
          pl.BlockSpec((K0, EB, D), lambda i: (0, i, 0)),
          pl.BlockSpec((EB, D), lambda i: (i, 0)),
          pl.BlockSpec((K0, EB, D), lambda i: (0, i, 0)),
          pl.BlockSpec((K0, EB, D), lambda i: (0, i, 0)),
      ] + [wspec] * 9 + [pl.BlockSpec((1, D), lambda i: (0, 0))],
      out_specs=pl.BlockSpec((EB, D), lambda i: (i, 0)),
      out_shape=jax.ShapeDtypeStruct((NE, D), jnp.float32),
  )(h0u, h1u, s2u, h0i, h1i, s2i, *weights, wlin_pad, blin)


def kernel(user_feat, item_feat, edge_feature, pos_src, pos_dst, neg_src,
           neg_dst, user_nb, item_nb, W_pu, b_pu, W_pi, b_pi, Wsu0, Wnu0,
           Wsu1, Wnu1, Wsi0, Wni0, Wsi1, Wni1, W_lin, b_lin):
  up = _project(user_feat, W_pu, b_pu)
  ip = _project(item_feat, W_pi, b_pi)
  B = pos_src.shape[0] + neg_src.shape[0]
  src = jnp.concatenate([pos_src, neg_src]).astype(jnp.int32)
  dst = jnp.concatenate([pos_dst, neg_dst]).astype(jnp.int32)
  srcp = jnp.pad(src, (0, NE - B))
  dstp = jnp.pad(dst, (0, NE - B))
  unbf = user_nb.astype(jnp.int32).reshape(-1)
  inbf = item_nb.astype(jnp.int32).reshape(-1)
  h0u, h1u, s2u, h0i, h1i, s2i = _sc_gather(up, ip, unbf, inbf, srcp, dstp)
  wlin_pad = jnp.pad(W_lin, ((0, 0), (0, D - 1)))
  blin = jnp.broadcast_to(b_lin.reshape(1, 1), (1, D))
  weights = (Wsu0, Wnu0, Wsu1, Wnu1, Wsi0, Wni0, Wsi1, Wni1)
  out = _gnn(h0u, h1u, s2u, h0i, h1i, s2i, weights, wlin_pad, blin)
  return out[:B, :1]


# speedup vs baseline: 3.6558x; 1.0012x over previous
import jax
import jax.numpy as jnp
from jax import lax
from jax.experimental import pallas as pl
from jax.experimental.pallas import tpu as pltpu
from jax.experimental.pallas import tpu_sc as plsc

K0 = 10
K1 = 5
D = 128
NC = 2
NS = 16
NW = NC * NS
NE = 4096
EPW = NE // NW
C = 32
NCH = EPW // C
NB1 = C * K0
NU2 = NB1 * K1
G = 80
NG = NU2 // G
MPG = G // K1
EB = 256
_SEG1 = ((0, 128), (128, 128), (256, 64))
_SEG2 = tuple((o * 128, 128) for o in range(12)) + ((1536, 64),)


def _proj_body(x_ref, w_ref, b_ref, o_ref):
  o_ref[...] = jax.nn.sigmoid(
      jnp.dot(x_ref[...], w_ref[...], preferred_element_type=jnp.float32)
      + b_ref[...])


def _project(x, W, b):
  N, Din = x.shape
  PD = W.shape[1]
  blk = 1000
  return pl.pallas_call(
      _proj_body,
      grid=(N // blk,),
      in_specs=[
          pl.BlockSpec((blk, Din), lambda i: (i, 0)),
          pl.BlockSpec((Din, PD), lambda i: (0, 0)),
          pl.BlockSpec((1, PD), lambda i: (0, 0)),
      ],
      out_specs=pl.BlockSpec((blk, PD), lambda i: (i, 0)),
      out_shape=jax.ShapeDtypeStruct((N, PD), jnp.float32),
  )(x, W, b.reshape(1, PD))


def _sc_body(up, ip, unbf, inbf, src, dst,
             h0u, h1u, s2u, h0i, h1i, s2i,
             srcv, idx1v, u1v, h0v, h1v, idx2v, u2v, h2a, h2b, m2v,
             semI, semF, semH, semA, semB):
  cid = lax.axis_index("c")
  sid = lax.axis_index("s")
  wid = cid * NS + sid
  iota = lax.iota(jnp.int32, 16)
  vK0 = jnp.full((16,), K0, jnp.int32)
  vK1 = jnp.full((16,), K1, jnp.int32)

  def accum(g, buf):
    def body(m, _):
      r = m * K1
      mrow = g * MPG + m
      for v in range(D // 16):
        sl = pl.ds(v * 16, 16)
        s = (buf[r, sl] + buf[r + 1, sl] + buf[r + 2, sl]
             + buf[r + 3, sl] + buf[r + 4, sl])
        m2v[mrow, sl] = s
      return _
    lax.fori_loop(0, MPG, body, None)

  for (ev, nbAf, nbBf, featA, featB, h0o, h1o, s2o) in (
      (src, unbf, inbf, up, ip, h0u, h1u, s2u),
      (dst, inbf, unbf, ip, up, h0i, h1i, s2i)):
    pltpu.sync_copy(ev.at[pl.ds(wid * EPW, EPW)], srcv)
    cp_h0 = pltpu.async_copy(featA.at[srcv], h0v, semH)

    def stage_xy(c):
      e0 = wid * EPW + c * C
      po = (c % 2) * NU2

      for i in range(NB1 // 16):
        k0v = i // 2
        sl = srcv[pl.ds(c * C + (i % 2) * 16, 16)]
        idx1v[pl.ds(i * 16, 16)] = sl * vK0 + jnp.full((16,), k0v, jnp.int32)
      cps = [pltpu.async_copy(nbAf.at[idx1v.at[pl.ds(o, n)]],
                              u1v.at[pl.ds(o, n)], semF) for (o, n) in _SEG1]
      if c == 0:
        cp_h0.wait()
        pltpu.sync_copy(h0v, h0o.at[pl.ds(wid * EPW, EPW)])
      for cp in cps:
        cp.wait()

      cpsF = [pltpu.async_copy(featB.at[u1v.at[pl.ds(o, n)]],
                               h1v.at[pl.ds(o, n)], semF) for (o, n) in _SEG1]

      def build_idx2(i, _):
        base = pl.multiple_of(i * 16, 16)
        q = base + iota
        p0 = base // 5
        v16 = u1v[pl.ds(p0, 16)]
        p = lax.div(q, vK1)
        d = p - jnp.full((16,), p0, jnp.int32)
        exp = jnp.where(
            d == 0, jnp.full((16,), v16[0], jnp.int32),
            jnp.where(d == 1, jnp.full((16,), v16[1], jnp.int32),
                      jnp.where(d == 2, jnp.full((16,), v16[2], jnp.int32),
                                jnp.full((16,), v16[3], jnp.int32))))
        idx2v[pl.ds(po + base, 16)] = exp * vK0 + (q - p * vK1)
        return _
      lax.fori_loop(0, NU2 // 16, build_idx2, None)
      for (o, n) in _SEG2:
        pltpu.async_copy(nbBf.at[idx2v.at[pl.ds(po + o, n)]],
                         u2v.at[pl.ds(po + o, n)], semI)
      for cp in cpsF:
        cp.wait()
      for k0 in range(K0):
        pltpu.sync_copy(h1v.at[pl.ds(k0 * C, C)], h1o.at[k0, pl.ds(e0, C)])

    def stage_z(c):
      e0 = wid * EPW + c * C
      po = (c % 2) * NU2
      for (o, n) in _SEG2:
        pltpu.make_async_copy(nbBf.at[idx2v.at[pl.ds(po + o, n)]],
                              u2v.at[pl.ds(po + o, n)], semI).wait()

      def issue_h2(g, buf, sem):
        gb = pl.multiple_of(g * G, G)
        return pltpu.async_copy(featA.at[u2v.at[pl.ds(po + gb, G)]], buf, sem)

      def wait_h2(g, buf, sem):
        gb = pl.multiple_of(g * G, G)
        pltpu.make_async_copy(featA.at[u2v.at[pl.ds(po + gb, G)]],
                              buf, sem).wait()

      issue_h2(0, h2a, semA)

      def pipe(t, _):
        g0 = 2 * t
        issue_h2(g0 + 1, h2b, semB)
        wait_h2(g0, h2a, semA)
        accum(g0, h2a)

        @pl.when(t < NG // 2 - 1)
        def _():
          issue_h2(g0 + 2, h2a, semA)
        wait_h2(g0 + 1, h2b, semB)
        accum(g0 + 1, h2b)
        return _
      lax.fori_loop(0, NG // 2, pipe, None)

      for k0 in range(K0):
        pltpu.sync_copy(m2v.at[pl.ds(k0 * C, C)], s2o.at[k0, pl.ds(e0, C)])

    stage_xy(0)
    stage_xy(1)
    stage_z(0)
    stage_xy(2)
    stage_z(1)
    stage_xy(3)
    stage_z(2)
    stage_z(3)


def _sc_gather(up, ip, unbf, inbf, srcp, dstp):
  mesh = plsc.VectorSubcoreMesh(core_axis_name="c", subcore_axis_name="s",
                                num_cores=NC, num_subcores=NS)
  edge_t = jax.ShapeDtypeStruct((NE, D), jnp.float32)
  hop_t = jax.ShapeDtypeStruct((K0, NE, D), jnp.float32)
  f = pl.kernel(
      _sc_body,
      out_type=[edge_t, hop_t, hop_t, edge_t, hop_t, hop_t],
      mesh=mesh,
      scratch_types=[
          pltpu.VMEM((EPW,), jnp.int32),
          pltpu.VMEM((NB1,), jnp.int32),
          pltpu.VMEM((NB1 + 16,), jnp.int32),
          pltpu.VMEM((EPW, D), jnp.float32),
          pltpu.VMEM((NB1, D), jnp.float32),
          pltpu.VMEM((2 * NU2,), jnp.int32),
          pltpu.VMEM((2 * NU2,), jnp.int32),
          pltpu.VMEM((G, D), jnp.float32),
          pltpu.VMEM((G, D), jnp.float32),
          pltpu.VMEM((NB1, D), jnp.float32),
          pltpu.SemaphoreType.DMA,
          pltpu.SemaphoreType.DMA,
          pltpu.SemaphoreType.DMA,
          pltpu.SemaphoreType.DMA,
          pltpu.SemaphoreType.DMA,
      ],
  )
  return f(up, ip, unbf, inbf, srcp, dstp)


def _gnn_body(h0u, h1u, s2u, h0i, h1i, s2i,
              wsu0, wnu0, wsu1, wnu1, wsi0, wni0, wsi1, wni1,
              wlin, blin, o_ref):
  def side(h0r, h1r, s2r, ws0, wn0, ws1, wn1):
    h0 = h0r[...]
    m1 = h1r[0]
    for k in range(1, K0):
      m1 = m1 + h1r[k]
    g0 = jax.nn.relu(
        jnp.dot(h0, ws0[...], preferred_element_type=jnp.float32)
        + jnp.dot(m1 * (1.0 / K0), wn0[...], preferred_element_type=jnp.float32))
    mg1 = None
    for k in range(K0):
      g1k = jax.nn.relu(
          jnp.dot(h1r[k], ws0[...], preferred_element_type=jnp.float32)
          + jnp.dot(s2r[k] * (1.0 / K1), wn0[...],
                    preferred_element_type=jnp.float32))
      mg1 = g1k if mg1 is None else mg1 + g1k
    return (jnp.dot(g0, ws1[...], preferred_element_type=jnp.float32)
            + jnp.dot(mg1 * (1.0 / K0), wn1[...],
                      preferred_element_type=jnp.float32))

  uh = side(h0u, h1u, s2u, wsu0, wnu0, wsu1, wnu1)
  ih = side(h0i, h1i, s2i, wsi0, wni0, wsi1, wni1)
  pred = uh * ih
  o_ref[...] = (jnp.dot(pred, wlin[...], preferred_element_type=jnp.float32)
                + blin[...])


def _gnn(h0u, h1u, s2u, h0i, h1i, s2i, weights, wlin_pad, blin):
  wspec = pl.BlockSpec((D, D), lambda i: (0, 0))
  return pl.pallas_call(
      _gnn_body,
      grid=(NE // EB,),
      in_specs=[
          pl.BlockSpec((EB, D), lambda i: (i, 0)),
          pl.BlockSpec((K0, EB, D), lambda i: (0, i, 0)),
          pl.BlockSpec((K0, EB, D), lambda i: (0, i, 0)),
          pl.BlockSpec((EB, D), lambda i: (i, 0)),
          pl.BlockSpec((K0, EB, D), lambda i: (0, i, 0)),
          pl.BlockSpec((K0, EB, D), lambda i: (0, i, 0)),
      ] + [wspec] * 9 + [pl.BlockSpec((1, D), lambda i: (0, 0))],
      out_specs=pl.BlockSpec((EB, D), lambda i: (i, 0)),
      out_shape=jax.ShapeDtypeStruct((NE, D), jnp.float32),
  )(h0u, h1u, s2u, h0i, h1i, s2i, *weights, wlin_pad, blin)


def kernel(user_feat, item_feat, edge_feature, pos_src, pos_dst, neg_src,
           neg_dst, user_nb, item_nb, W_pu, b_pu, W_pi, b_pi, Wsu0, Wnu0,
           Wsu1, Wnu1, Wsi0, Wni0, Wsi1, Wni1, W_lin, b_lin):
  up = _project(user_feat, W_pu, b_pu)
  ip = _project(item_feat, W_pi, b_pi)
  B = pos_src.shape[0] + neg_src.shape[0]
  src = jnp.concatenate([pos_src, neg_src]).astype(jnp.int32)
  dst = jnp.concatenate([pos_dst, neg_dst]).astype(jnp.int32)
  srcp = jnp.pad(src, (0, NE - B))
  dstp = jnp.pad(dst, (0, NE - B))
  unbf = user_nb.astype(jnp.int32).reshape(-1)
  inbf = item_nb.astype(jnp.int32).reshape(-1)
  h0u, h1u, s2u, h0i, h1i, s2i = _sc_gather(up, ip, unbf, inbf, srcp, dstp)
  wlin_pad = jnp.pad(W_lin, ((0, 0), (0, D - 1)))
  blin = jnp.broadcast_to(b_lin.reshape(1, 1), (1, D))
  weights = (Wsu0, Wnu0, Wsu1, Wnu1, Wsi0, Wni0, Wsi1, Wni1)
  out = _gnn(h0u, h1u, s2u, h0i, h1i, s2i, weights, wlin_pad, blin)
  return out[:B, :1]


# speedup vs baseline: 3.6576x; 1.0005x over previous
import jax
import jax.numpy as jnp
from jax import lax
from jax.experimental import pallas as pl
from jax.experimental.pallas import tpu as pltpu
from jax.experimental.pallas import tpu_sc as plsc

K0 = 10
K1 = 5
D = 128
NC = 2
NS = 16
NW = NC * NS
NE = 4096
EPW = NE // NW
C = 32
NCH = EPW // C
NB1 = C * K0
NU2 = NB1 * K1
G = 80
NG = NU2 // G
MPG = G // K1
EB = 256
_SEG1 = ((0, 128), (128, 128), (256, 64))
_SEG2 = tuple((o * 128, 128) for o in range(12)) + ((1536, 64),)


def _proj_body(x_ref, w_ref, b_ref, o_ref):
  o_ref[...] = jax.nn.sigmoid(
      jnp.dot(x_ref[...], w_ref[...], preferred_element_type=jnp.float32)
      + b_ref[...])


def _project(x, W, b):
  N, Din = x.shape
  PD = W.shape[1]
  blk = 1000
  return pl.pallas_call(
      _proj_body,
      grid=(N // blk,),
      in_specs=[
          pl.BlockSpec((blk, Din), lambda i: (i, 0)),
          pl.BlockSpec((Din, PD), lambda i: (0, 0)),
          pl.BlockSpec((1, PD), lambda i: (0, 0)),
      ],
      out_specs=pl.BlockSpec((blk, PD), lambda i: (i, 0)),
      out_shape=jax.ShapeDtypeStruct((N, PD), jnp.float32),
  )(x, W, b.reshape(1, PD))


def _sc_body(up, ip, unbf, inbf, src, dst,
             h0u, h1u, s2u, h0i, h1i, s2i,
             srcv, idx1v, u1v, h0v, h1v, idx2v, u2v, h2a, h2b, m2v,
             semI, semF, semH, semA, semB):
  cid = lax.axis_index("c")
  sid = lax.axis_index("s")
  wid = cid * NS + sid
  iota = lax.iota(jnp.int32, 16)
  vK0 = jnp.full((16,), K0, jnp.int32)
  vK1 = jnp.full((16,), K1, jnp.int32)

  def accum(g, buf):
    def body(m, _):
      r = m * K1
      mrow = g * MPG + m
      for v in range(D // 16):
        sl = pl.ds(v * 16, 16)
        s = (buf[r, sl] + buf[r + 1, sl] + buf[r + 2, sl]
             + buf[r + 3, sl] + buf[r + 4, sl])
        m2v[mrow, sl] = s
      return _
    lax.fori_loop(0, MPG, body, None)

  for (ev, nbAf, nbBf, featA, featB, h0o, h1o, s2o) in (
      (src, unbf, inbf, up, ip, h0u, h1u, s2u),
      (dst, inbf, unbf, ip, up, h0i, h1i, s2i)):
    pltpu.sync_copy(ev.at[pl.ds(wid * EPW, EPW)], srcv)
    cp_h0 = pltpu.async_copy(featA.at[srcv], h0v, semH)

    def stage_xy(c):
      e0 = wid * EPW + c * C
      po = (c % 2) * NU2

      with jax.named_scope("xy_idx1"):
        for i in range(NB1 // 16):
          k0v = i // 2
          sl = srcv[pl.ds(c * C + (i % 2) * 16, 16)]
          idx1v[pl.ds(i * 16, 16)] = sl * vK0 + jnp.full((16,), k0v, jnp.int32)
        cps = [pltpu.async_copy(nbAf.at[idx1v.at[pl.ds(o, n)]],
                                u1v.at[pl.ds(o, n)], semF) for (o, n) in _SEG1]
        if c == 0:
          cp_h0.wait()
          pltpu.sync_copy(h0v, h0o.at[pl.ds(wid * EPW, EPW)])
        for cp in cps:
          cp.wait()

      cpsF = [pltpu.async_copy(featB.at[u1v.at[pl.ds(o, n)]],
                               h1v.at[pl.ds(o, n)], semF) for (o, n) in _SEG1]

      def build_idx2(i, _):
        base = pl.multiple_of(i * 16, 16)
        q = base + iota
        p0 = base // 5
        v16 = u1v[pl.ds(p0, 16)]
        p = lax.div(q, vK1)
        d = p - jnp.full((16,), p0, jnp.int32)
        exp = jnp.where(
            d == 0, jnp.full((16,), v16[0], jnp.int32),
            jnp.where(d == 1, jnp.full((16,), v16[1], jnp.int32),
                      jnp.where(d == 2, jnp.full((16,), v16[2], jnp.int32),
                                jnp.full((16,), v16[3], jnp.int32))))
        idx2v[pl.ds(po + base, 16)] = exp * vK0 + (q - p * vK1)
        return _
      with jax.named_scope("xy_idx2"):
        lax.fori_loop(0, NU2 // 16, build_idx2, None)
        for (o, n) in _SEG2:
          pltpu.async_copy(nbBf.at[idx2v.at[pl.ds(po + o, n)]],
                           u2v.at[pl.ds(po + o, n)], semI)
      with jax.named_scope("xy_h1wr"):
        for cp in cpsF:
          cp.wait()
        for k0 in range(K0):
          pltpu.sync_copy(h1v.at[pl.ds(k0 * C, C)], h1o.at[k0, pl.ds(e0, C)])

    def stage_z(c):
      e0 = wid * EPW + c * C
      po = (c % 2) * NU2
      with jax.named_scope("z_u2wait"):
        for (o, n) in _SEG2:
          pltpu.make_async_copy(nbBf.at[idx2v.at[pl.ds(po + o, n)]],
                                u2v.at[pl.ds(po + o, n)], semI).wait()

      def issue_h2(g, buf, sem):
        gb = pl.multiple_of(g * G, G)
        return pltpu.async_copy(featA.at[u2v.at[pl.ds(po + gb, G)]], buf, sem)

      def wait_h2(g, buf, sem):
        gb = pl.multiple_of(g * G, G)
        pltpu.make_async_copy(featA.at[u2v.at[pl.ds(po + gb, G)]],
                              buf, sem).wait()

      issue_h2(0, h2a, semA)

      def pipe(t, _):
        g0 = 2 * t
        issue_h2(g0 + 1, h2b, semB)
        wait_h2(g0, h2a, semA)
        accum(g0, h2a)

        @pl.when(t < NG // 2 - 1)
        def _():
          issue_h2(g0 + 2, h2a, semA)
        wait_h2(g0 + 1, h2b, semB)
        accum(g0 + 1, h2b)
        return _
      with jax.named_scope("z_pipe"):
        lax.fori_loop(0, NG // 2, pipe, None)

      with jax.named_scope("z_s2wr"):
        for k0 in range(K0):
          pltpu.sync_copy(m2v.at[pl.ds(k0 * C, C)], s2o.at[k0, pl.ds(e0, C)])

    stage_xy(0)
    stage_xy(1)
    stage_z(0)
    stage_xy(2)
    stage_z(1)
    stage_xy(3)
    stage_z(2)
    stage_z(3)


def _sc_gather(up, ip, unbf, inbf, srcp, dstp):
  mesh = plsc.VectorSubcoreMesh(core_axis_name="c", subcore_axis_name="s",
                                num_cores=NC, num_subcores=NS)
  edge_t = jax.ShapeDtypeStruct((NE, D), jnp.float32)
  hop_t = jax.ShapeDtypeStruct((K0, NE, D), jnp.float32)
  f = pl.kernel(
      _sc_body,
      out_type=[edge_t, hop_t, hop_t, edge_t, hop_t, hop_t],
      mesh=mesh,
      scratch_types=[
          pltpu.VMEM((EPW,), jnp.int32),
          pltpu.VMEM((NB1,), jnp.int32),
          pltpu.VMEM((NB1 + 16,), jnp.int32),
          pltpu.VMEM((EPW, D), jnp.float32),
          pltpu.VMEM((NB1, D), jnp.float32),
          pltpu.VMEM((2 * NU2,), jnp.int32),
          pltpu.VMEM((2 * NU2,), jnp.int32),
          pltpu.VMEM((G, D), jnp.float32),
          pltpu.VMEM((G, D), jnp.float32),
          pltpu.VMEM((NB1, D), jnp.float32),
          pltpu.SemaphoreType.DMA,
          pltpu.SemaphoreType.DMA,
          pltpu.SemaphoreType.DMA,
          pltpu.SemaphoreType.DMA,
          pltpu.SemaphoreType.DMA,
      ],
  )
  return f(up, ip, unbf, inbf, srcp, dstp)


def _gnn_body(h0u, h1u, s2u, h0i, h1i, s2i,
              wsu0, wnu0, wsu1, wnu1, wsi0, wni0, wsi1, wni1,
              wlin, blin, o_ref):
  def side(h0r, h1r, s2r, ws0, wn0, ws1, wn1):
    h0 = h0r[...]
    m1 = h1r[0]
    for k in range(1, K0):
      m1 = m1 + h1r[k]
    g0 = jax.nn.relu(
        jnp.dot(h0, ws0[...], preferred_element_type=jnp.float32)
        + jnp.dot(m1 * (1.0 / K0), wn0[...], preferred_element_type=jnp.float32))
    mg1 = None
    for k in range(K0):
      g1k = jax.nn.relu(
          jnp.dot(h1r[k], ws0[...], preferred_element_type=jnp.float32)
          + jnp.dot(s2r[k] * (1.0 / K1), wn0[...],
                    preferred_element_type=jnp.float32))
      mg1 = g1k if mg1 is None else mg1 + g1k
    return (jnp.dot(g0, ws1[...], preferred_element_type=jnp.float32)
            + jnp.dot(mg1 * (1.0 / K0), wn1[...],
                      preferred_element_type=jnp.float32))

  uh = side(h0u, h1u, s2u, wsu0, wnu0, wsu1, wnu1)
  ih = side(h0i, h1i, s2i, wsi0, wni0, wsi1, wni1)
  pred = uh * ih
  o_ref[...] = (jnp.dot(pred, wlin[...], preferred_element_type=jnp.float32)
                + blin[...])


def _gnn(h0u, h1u, s2u, h0i, h1i, s2i, weights, wlin_pad, blin):
  wspec = pl.BlockSpec((D, D), lambda i: (0, 0))
  return pl.pallas_call(
      _gnn_body,
      grid=(NE // EB,),
      in_specs=[
          pl.BlockSpec((EB, D), lambda i: (i, 0)),
          pl.BlockSpec((K0, EB, D), lambda i: (0, i, 0)),
          pl.BlockSpec((K0, EB, D), lambda i: (0, i, 0)),
          pl.BlockSpec((EB, D), lambda i: (i, 0)),
          pl.BlockSpec((K0, EB, D), lambda i: (0, i, 0)),
          pl.BlockSpec((K0, EB, D), lambda i: (0, i, 0)),
      ] + [wspec] * 9 + [pl.BlockSpec((1, D), lambda i: (0, 0))],
      out_specs=pl.BlockSpec((EB, D), lambda i: (i, 0)),
      out_shape=jax.ShapeDtypeStruct((NE, D), jnp.float32),
  )(h0u, h1u, s2u, h0i, h1i, s2i, *weights, wlin_pad, blin)


def kernel(user_feat, item_feat, edge_feature, pos_src, pos_dst, neg_src,
           neg_dst, user_nb, item_nb, W_pu, b_pu, W_pi, b_pi, Wsu0, Wnu0,
           Wsu1, Wnu1, Wsi0, Wni0, Wsi1, Wni1, W_lin, b_lin):
  up = _project(user_feat, W_pu, b_pu)
  ip = _project(item_feat, W_pi, b_pi)
  B = pos_src.shape[0] + neg_src.shape[0]
  src = jnp.concatenate([pos_src, neg_src]).astype(jnp.int32)
  dst = jnp.concatenate([pos_dst, neg_dst]).astype(jnp.int32)
  srcp = jnp.pad(src, (0, NE - B))
  dstp = jnp.pad(dst, (0, NE - B))
  unbf = user_nb.astype(jnp.int32).reshape(-1)
  inbf = item_nb.astype(jnp.int32).reshape(-1)
  h0u, h1u, s2u, h0i, h1i, s2i = _sc_gather(up, ip, unbf, inbf, srcp, dstp)
  wlin_pad = jnp.pad(W_lin, ((0, 0), (0, D - 1)))
  blin = jnp.broadcast_to(b_lin.reshape(1, 1), (1, D))
  weights = (Wsu0, Wnu0, Wsu1, Wnu1, Wsi0, Wni0, Wsi1, Wni1)
  out = _gnn(h0u, h1u, s2u, h0i, h1i, s2i, weights, wlin_pad, blin)
  return out[:B, :1]


# speedup vs baseline: 4.2814x; 1.1706x over previous
import jax
import jax.numpy as jnp
from jax import lax
from jax.experimental import pallas as pl
from jax.experimental.pallas import tpu as pltpu
from jax.experimental.pallas import tpu_sc as plsc

K0 = 10
K1 = 5
D = 128
NC = 2
NS = 16
NW = NC * NS
NE = 4096
EPW = NE // NW
C = 32
NCH = EPW // C
NB1 = C * K0
NU2 = NB1 * K1
G = 80
NG = NU2 // G
MPG = G // K1
EB = 256
_SEG1 = ((0, 128), (128, 128), (256, 64))
_SEG2 = tuple((o * 128, 128) for o in range(12)) + ((1536, 64),)


def _proj_body(x_ref, w_ref, b_ref, o_ref):
  o_ref[...] = jax.nn.sigmoid(
      jnp.dot(x_ref[...], w_ref[...], preferred_element_type=jnp.float32)
      + b_ref[...])


def _project(x, W, b):
  N, Din = x.shape
  PD = W.shape[1]
  blk = 2000
  return pl.pallas_call(
      _proj_body,
      grid=(N // blk,),
      in_specs=[
          pl.BlockSpec((blk, Din), lambda i: (i, 0)),
          pl.BlockSpec((Din, PD), lambda i: (0, 0)),
          pl.BlockSpec((1, PD), lambda i: (0, 0)),
      ],
      out_specs=pl.BlockSpec((blk, PD), lambda i: (i, 0)),
      out_shape=jax.ShapeDtypeStruct((N, PD), jnp.float32),
  )(x, W, b.reshape(1, PD))


def _sc_body(up, ip, unbf, inbf, src, dst,
             h0u, h1u, s2u, h0i, h1i, s2i,
             srcv, idx1v, u1v, h1v, idx2v, u2v, h2a, h2b, h2c, h2d, m2v,
             semI, semF, semH, semA, semB, semC, semD, semW):
  cid = lax.axis_index("c")
  sid = lax.axis_index("s")
  wid = cid * NS + sid
  iota = lax.iota(jnp.int32, 16)
  vK0 = jnp.full((16,), K0, jnp.int32)
  vK1 = jnp.full((16,), K1, jnp.int32)
  h2bufs = (h2a, h2b, h2c, h2d)
  h2sems = (semA, semB, semC, semD)
  prev_wr = []

  def accum(g, buf):
    def body(m, _):
      r = m * K1
      mrow = g * MPG + m
      for v in range(D // 16):
        sl = pl.ds(v * 16, 16)
        s = (buf[r, sl] + buf[r + 1, sl] + buf[r + 2, sl]
             + buf[r + 3, sl] + buf[r + 4, sl])
        m2v[mrow, sl] = s
      return _
    lax.fori_loop(0, MPG, body, None)

  def drain_s2():
    if prev_wr:
      e0p, s2op = prev_wr.pop()
      for k0 in range(K0):
        pltpu.make_async_copy(m2v.at[pl.ds(k0 * C, C)],
                              s2op.at[k0, pl.ds(e0p, C)], semW).wait()

  for (ev, nbAf, nbBf, featA, featB, h0o, h1o, s2o) in (
      (src, unbf, inbf, up, ip, h0u, h1u, s2u),
      (dst, inbf, unbf, ip, up, h0i, h1i, s2i)):
    pltpu.sync_copy(ev.at[pl.ds(wid * EPW, EPW)], srcv)
    for c in range(NCH):
      e0 = wid * EPW + c * C
      idx_c = srcv.at[pl.ds(c * C, C)]
      h0v = h2a.at[pl.ds(0, C)]
      cp_h0 = pltpu.async_copy(featA.at[idx_c], h0v, semH)

      for i in range(NB1 // 16):
        k0v = i // 2
        sl = srcv[pl.ds(c * C + (i % 2) * 16, 16)]
        idx1v[pl.ds(i * 16, 16)] = sl * vK0 + jnp.full((16,), k0v, jnp.int32)
      cps = [pltpu.async_copy(nbAf.at[idx1v.at[pl.ds(o, n)]],
                              u1v.at[pl.ds(o, n)], semF) for (o, n) in _SEG1]
      cp_h0.wait()
      pltpu.sync_copy(h0v, h0o.at[pl.ds(e0, C)])
      for cp in cps:
        cp.wait()

      cpsF = [pltpu.async_copy(featB.at[u1v.at[pl.ds(o, n)]],
                               h1v.at[pl.ds(o, n)], semF) for (o, n) in _SEG1]

      def build_idx2(i, _):
        base = pl.multiple_of(i * 16, 16)
        q = base + iota
        p0 = base // 5
        v16 = u1v[pl.ds(p0, 16)]
        p = lax.div(q, vK1)
        d = p - jnp.full((16,), p0, jnp.int32)
        exp = jnp.where(
            d == 0, jnp.full((16,), v16[0], jnp.int32),
            jnp.where(d == 1, jnp.full((16,), v16[1], jnp.int32),
                      jnp.where(d == 2, jnp.full((16,), v16[2], jnp.int32),
                                jnp.full((16,), v16[3], jnp.int32))))
        idx2v[pl.ds(base, 16)] = exp * vK0 + (q - p * vK1)
        return _
      lax.fori_loop(0, NU2 // 16, build_idx2, None)
      cpsU = [pltpu.async_copy(nbBf.at[idx2v.at[pl.ds(o, n)]],
                               u2v.at[pl.ds(o, n)], semI) for (o, n) in _SEG2]
      for cp in cpsF:
        cp.wait()
      for k0 in range(K0):
        pltpu.sync_copy(h1v.at[pl.ds(k0 * C, C)], h1o.at[k0, pl.ds(e0, C)])
      for cp in cpsU:
        cp.wait()

      def issue_h2(g, buf, sem):
        gb = pl.multiple_of(g * G, G)
        return pltpu.async_copy(featA.at[u2v.at[pl.ds(gb, G)]], buf, sem)

      def wait_h2(g, buf, sem):
        gb = pl.multiple_of(g * G, G)
        pltpu.make_async_copy(featA.at[u2v.at[pl.ds(gb, G)]], buf, sem).wait()

      for j in range(4):
        issue_h2(j, h2bufs[j], h2sems[j])
      drain_s2()

      def pipe(t, _):
        gb = 4 * t
        for j in range(4):
          wait_h2(gb + j, h2bufs[j], h2sems[j])
          accum(gb + j, h2bufs[j])

          @pl.when(gb + j + 4 < NG)
          def _():
            issue_h2(gb + j + 4, h2bufs[j], h2sems[j])
        return _
      lax.fori_loop(0, NG // 4, pipe, None)

      for k0 in range(K0):
        pltpu.async_copy(m2v.at[pl.ds(k0 * C, C)],
                         s2o.at[k0, pl.ds(e0, C)], semW)
      prev_wr.append((e0, s2o))
  drain_s2()


def _sc_gather(up, ip, unbf, inbf, srcp, dstp):
  mesh = plsc.VectorSubcoreMesh(core_axis_name="c", subcore_axis_name="s",
                                num_cores=NC, num_subcores=NS)
  edge_t = jax.ShapeDtypeStruct((NE, D), jnp.float32)
  hop_t = jax.ShapeDtypeStruct((K0, NE, D), jnp.float32)
  f = pl.kernel(
      _sc_body,
      out_type=[edge_t, hop_t, hop_t, edge_t, hop_t, hop_t],
      mesh=mesh,
      scratch_types=[
          pltpu.VMEM((EPW,), jnp.int32),
          pltpu.VMEM((NB1,), jnp.int32),
          pltpu.VMEM((NB1 + 16,), jnp.int32),
          pltpu.VMEM((NB1, D), jnp.float32),
          pltpu.VMEM((NU2,), jnp.int32),
          pltpu.VMEM((NU2,), jnp.int32),
          pltpu.VMEM((G, D), jnp.float32),
          pltpu.VMEM((G, D), jnp.float32),
          pltpu.VMEM((G, D), jnp.float32),
          pltpu.VMEM((G, D), jnp.float32),
          pltpu.VMEM((NB1, D), jnp.float32),
          pltpu.SemaphoreType.DMA,
          pltpu.SemaphoreType.DMA,
          pltpu.SemaphoreType.DMA,
          pltpu.SemaphoreType.DMA,
          pltpu.SemaphoreType.DMA,
          pltpu.SemaphoreType.DMA,
          pltpu.SemaphoreType.DMA,
          pltpu.SemaphoreType.DMA,
      ],
  )
  return f(up, ip, unbf, inbf, srcp, dstp)


def _gnn_body(h0u, h1u, s2u, h0i, h1i, s2i,
              wsu0, wnu0, wsu1, wnu1, wsi0, wni0, wsi1, wni1,
              wlin, blin, o_ref):
  def side(h0r, h1r, s2r, ws0, wn0, ws1, wn1):
    h0 = h0r[...]
    m1 = h1r[0]
    for k in range(1, K0):
      m1 = m1 + h1r[k]
    g0 = jax.nn.relu(
        jnp.dot(h0, ws0[...], preferred_element_type=jnp.float32)
        + jnp.dot(m1 * (1.0 / K0), wn0[...], preferred_element_type=jnp.float32))
    mg1 = None
    for k in range(K0):
      g1k = jax.nn.relu(
          jnp.dot(h1r[k], ws0[...], preferred_element_type=jnp.float32)
          + jnp.dot(s2r[k] * (1.0 / K1), wn0[...],
                    preferred_element_type=jnp.float32))
      mg1 = g1k if mg1 is None else mg1 + g1k
    return (jnp.dot(g0, ws1[...], preferred_element_type=jnp.float32)
            + jnp.dot(mg1 * (1.0 / K0), wn1[...],
                      preferred_element_type=jnp.float32))

  uh = side(h0u, h1u, s2u, wsu0, wnu0, wsu1, wnu1)
  ih = side(h0i, h1i, s2i, wsi0, wni0, wsi1, wni1)
  pred = uh * ih
  o_ref[...] = (jnp.dot(pred, wlin[...], preferred_element_type=jnp.float32)
                + blin[...])


def _gnn(h0u, h1u, s2u, h0i, h1i, s2i, weights, wlin_pad, blin):
  wspec = pl.BlockSpec((D, D), lambda i: (0, 0))
  return pl.pallas_call(
      _gnn_body,
      grid=(NE // EB,),
      in_specs=[
          pl.BlockSpec((EB, D), lambda i: (i, 0)),
          pl.BlockSpec((K0, EB, D), lambda i: (0, i, 0)),
          pl.BlockSpec((K0, EB, D), lambda i: (0, i, 0)),
          pl.BlockSpec((EB, D), lambda i: (i, 0)),
          pl.BlockSpec((K0, EB, D), lambda i: (0, i, 0)),
          pl.BlockSpec((K0, EB, D), lambda i: (0, i, 0)),
      ] + [wspec] * 9 + [pl.BlockSpec((1, D), lambda i: (0, 0))],
      out_specs=pl.BlockSpec((EB, D), lambda i: (i, 0)),
      out_shape=jax.ShapeDtypeStruct((NE, D), jnp.float32),
  )(h0u, h1u, s2u, h0i, h1i, s2i, *weights, wlin_pad, blin)


def kernel(user_feat, item_feat, edge_feature, pos_src, pos_dst, neg_src,
           neg_dst, user_nb, item_nb, W_pu, b_pu, W_pi, b_pi, Wsu0, Wnu0,
           Wsu1, Wnu1, Wsi0, Wni0, Wsi1, Wni1, W_lin, b_lin):
  up = _project(user_feat, W_pu, b_pu)
  ip = _project(item_feat, W_pi, b_pi)
  B = pos_src.shape[0] + neg_src.shape[0]
  src = jnp.concatenate([pos_src, neg_src]).astype(jnp.int32)
  dst = jnp.concatenate([pos_dst, neg_dst]).astype(jnp.int32)
  srcp = jnp.pad(src, (0, NE - B))
  dstp = jnp.pad(dst, (0, NE - B))
  unbf = user_nb.astype(jnp.int32).reshape(-1)
  inbf = item_nb.astype(jnp.int32).reshape(-1)
  h0u, h1u, s2u, h0i, h1i, s2i = _sc_gather(up, ip, unbf, inbf, srcp, dstp)
  wlin_pad = jnp.pad(W_lin, ((0, 0), (0, D - 1)))
  blin = jnp.broadcast_to(b_lin.reshape(1, 1), (1, D))
  weights = (Wsu0, Wnu0, Wsu1, Wnu1, Wsi0, Wni0, Wsi1, Wni1)
  out = _gnn(h0u, h1u, s2u, h0i, h1i, s2i, weights, wlin_pad, blin)
  return out[:B, :1]
